# baseline (speedup 1.0000x reference)
import functools

import jax
import jax.numpy as jnp
from jax import lax
from jax.experimental import pallas as pl
from jax.experimental.pallas import tpu as pltpu
from jax.experimental.pallas import tpu_sc as plsc

_N = 10000
_E = 320000
_NHID = 128
_NCLASS = 64

_CH = 128
_NCHUNK = _E // _CH
_NTILES = 32
_NPAD = 10240
_ROWS_PER_TILE = _NPAD // 16
_BLK = 1000


_NCT = 80
_EPAD = _NTILES * _NCT * _CH


def _spmm_body(support_hbm, src_hbm, dst_hbm, out_hbm,
               src2d, dst2d, rows_v, acc_sh, gsem):
    cid = lax.axis_index("c")
    sid = lax.axis_index("s")
    wid = sid * 2 + cid

    def zfill(i, carry):
        r = i // (_NHID // 16)
        c = (i % (_NHID // 16)) * 16
        rows_v[r, pl.ds(c, 16)] = jnp.zeros((16,), jnp.float32)
        return carry
    lax.fori_loop(0, _CH * (_NHID // 16), zfill, 0)

    base_r = sid * _ROWS_PER_TILE
    def zacc(k, carry):
        pltpu.sync_copy(rows_v, acc_sh.at[pl.ds(base_r + k * _CH, _CH)])
        return carry
    lax.fori_loop(0, _ROWS_PER_TILE // _CH, zacc, 0)

    plsc.subcore_barrier()

    base_e = wid * _NCT * _CH

    def body(k, carry):
        e0 = pl.multiple_of(base_e + k * _CH, 8)
        pltpu.sync_copy(src_hbm.at[pl.ds(e0, _CH)], src2d)
        pltpu.sync_copy(dst_hbm.at[pl.ds(e0, _CH)], dst2d)
        pltpu.async_copy(support_hbm.at[src2d], rows_v, gsem).wait()
        pltpu.sync_copy(rows_v, acc_sh.at[dst2d], add=True)
        return carry
    lax.fori_loop(0, _NCT, body, 0)
    plsc.subcore_barrier()

    def ex(k, carry):
        r0 = base_r + k * _CH
        pltpu.sync_copy(acc_sh.at[pl.ds(r0, _CH)], rows_v)
        pltpu.sync_copy(rows_v, out_hbm.at[cid, pl.ds(r0, _CH)])
        return carry
    lax.fori_loop(0, _ROWS_PER_TILE // _CH, ex, 0)


@functools.cache
def _make_spmm():
    return pl.kernel(
        _spmm_body,
        out_type=jax.ShapeDtypeStruct((2, _NPAD, _NHID), jnp.float32),
        mesh=plsc.VectorSubcoreMesh(core_axis_name="c", subcore_axis_name="s"),
        scratch_types=[
            pltpu.VMEM((_CH,), jnp.int32),
            pltpu.VMEM((_CH,), jnp.int32),
            pltpu.VMEM((_CH, _NHID), jnp.float32),
            pltpu.VMEM_SHARED((_NPAD, _NHID), jnp.float32),
            pltpu.SemaphoreType.DMA,
        ],
    )


def _spmm(support, src, dst):
    npad_e = _EPAD - _E
    src_p = jnp.concatenate([src, jnp.zeros((npad_e,), jnp.int32)])
    dst_p = jnp.concatenate(
        [dst, _N + (jnp.arange(npad_e, dtype=jnp.int32) % (_NPAD - _N))])
    return _make_spmm()(support, src_p, dst_p)


def _dense_in_body(x_ref, w0_ref, wci_ref, bci_ref, sup_ref, ci_ref):
    x = x_ref[...]
    sup_ref[...] = jnp.dot(x, w0_ref[...], preferred_element_type=jnp.float32)
    ci_ref[...] = (jnp.dot(x, wci_ref[...], preferred_element_type=jnp.float32)
                   + bci_ref[...])


def _dense_in(x, W0, Wci, bci):
    full = pl.BlockSpec((_NHID, _NHID), lambda i: (0, 0))
    row = pl.BlockSpec((1, _NHID), lambda i: (0, 0))
    blk = pl.BlockSpec((_BLK, _NHID), lambda i: (i, 0))
    return pl.pallas_call(
        _dense_in_body,
        grid=(_N // _BLK,),
        in_specs=[blk, full, full, row],
        out_specs=[blk, blk],
        out_shape=[jax.ShapeDtypeStruct((_N, _NHID), jnp.float32),
                   jax.ShapeDtypeStruct((_N, _NHID), jnp.float32)],
    )(x, W0, Wci, bci)


def _gate(agg_a, agg_b, b, ci, x, wco, bco):
    out_x = agg_a[0] + agg_b[0] + b
    z = jax.nn.sigmoid(
        ci + jnp.dot(out_x, wco, preferred_element_type=jnp.float32) + bco)
    return z * out_x + (1.0 - z) * x


def _gate_next_body(agga_ref, aggb_ref, b_ref, ci_ref, x_ref, wco_ref,
                    bco_ref, wn_ref, out_ref):
    h = jax.nn.relu(_gate(agga_ref[...], aggb_ref[...], b_ref[...],
                          ci_ref[...], x_ref[...], wco_ref[...], bco_ref[...]))
    out_ref[...] = jnp.dot(h, wn_ref[...], preferred_element_type=jnp.float32)


def _gate_next(agg, b, ci, x, Wco, bco, Wnext):
    full = pl.BlockSpec((_NHID, _NHID), lambda i: (0, 0))
    row = pl.BlockSpec((1, _NHID), lambda i: (0, 0))
    blk = pl.BlockSpec((_BLK, _NHID), lambda i: (i, 0))
    agg0 = pl.BlockSpec((1, _BLK, _NHID), lambda i: (0, i, 0))
    agg1 = pl.BlockSpec((1, _BLK, _NHID), lambda i: (1, i, 0))
    return pl.pallas_call(
        _gate_next_body,
        grid=(_N // _BLK,),
        in_specs=[agg0, agg1, row, blk, blk, full, row, full],
        out_specs=blk,
        out_shape=jax.ShapeDtypeStruct((_N, _NHID), jnp.float32),
    )(agg, agg, b, ci, x, Wco, bco, Wnext)


def _gate_final_body(agga_ref, aggb_ref, b_ref, ci_ref, x_ref, wco_ref,
                     bco_ref, wf_ref, bf_ref, out_ref):
    h = _gate(agga_ref[...], aggb_ref[...], b_ref[...],
              ci_ref[...], x_ref[...], wco_ref[...], bco_ref[...])
    out_ref[...] = (jnp.dot(h, wf_ref[...], preferred_element_type=jnp.float32)
                    + bf_ref[...])


def _gate_final(agg, b, ci, x, Wco, bco, Wf, bf):
    full = pl.BlockSpec((_NHID, _NHID), lambda i: (0, 0))
    wf_spec = pl.BlockSpec((_NHID, _NCLASS), lambda i: (0, 0))
    row = pl.BlockSpec((1, _NHID), lambda i: (0, 0))
    rowf = pl.BlockSpec((1, _NCLASS), lambda i: (0, 0))
    blk = pl.BlockSpec((_BLK, _NHID), lambda i: (i, 0))
    blkf = pl.BlockSpec((_BLK, _NCLASS), lambda i: (i, 0))
    agg0 = pl.BlockSpec((1, _BLK, _NHID), lambda i: (0, i, 0))
    agg1 = pl.BlockSpec((1, _BLK, _NHID), lambda i: (1, i, 0))
    return pl.pallas_call(
        _gate_final_body,
        grid=(_N // _BLK,),
        in_specs=[agg0, agg1, row, blk, blk, full, row, wf_spec, rowf],
        out_specs=blkf,
        out_shape=jax.ShapeDtypeStruct((_N, _NCLASS), jnp.float32),
    )(agg, agg, b, ci, x, Wco, bco, Wf, bf)


def kernel(x, edge_index, W0, b0, W1, b1, W2, b2, Wci, bci, Wco, bco, Wf, bf):
    src = edge_index[0].astype(jnp.int32)
    dst = edge_index[1].astype(jnp.int32)
    bci2 = bci.reshape(1, _NHID)
    bco2 = bco.reshape(1, _NHID)
    bf2 = bf.reshape(1, _NCLASS)

    support0, ci = _dense_in(x, W0, Wci, bci2)
    agg0 = _spmm(support0, src, dst)
    support1 = _gate_next(agg0, b0, ci, x, Wco, bco2, W1)
    agg1 = _spmm(support1, src, dst)
    support2 = _gate_next(agg1, b1, ci, x, Wco, bco2, W2)
    agg2 = _spmm(support2, src, dst)
    return _gate_final(agg2, b2, ci, x, Wco, bco2, Wf, bf2)


# speedup vs baseline: 2.6262x; 2.6262x over previous
import functools

import jax
import jax.numpy as jnp
from jax import lax
from jax.experimental import pallas as pl
from jax.experimental.pallas import tpu as pltpu
from jax.experimental.pallas import tpu_sc as plsc

_N = 10000
_E = 320000
_NHID = 128
_NCLASS = 64

_CH = 128
_NCHUNK = _E // _CH
_NTILES = 32
_NPAD = 10240
_ROWS_PER_TILE = _NPAD // 16
_BLK = 1000


_NCT = 80
_EPAD = _NTILES * _NCT * _CH


def _spmm_body(support_hbm, src_hbm, dst_hbm, out_hbm,
               src2d, dst2d, rows_v, acc_sh, gsem):
    cid = lax.axis_index("c")
    sid = lax.axis_index("s")
    wid = sid * 2 + cid

    def zfill(i, carry):
        r = i // (_NHID // 16)
        c = (i % (_NHID // 16)) * 16
        rows_v[r, pl.ds(c, 16)] = jnp.zeros((16,), jnp.float32)
        return carry
    lax.fori_loop(0, _CH * (_NHID // 16), zfill, 0)

    base_r = sid * _ROWS_PER_TILE
    def zacc(k, carry):
        pltpu.sync_copy(rows_v, acc_sh.at[pl.ds(base_r + k * _CH, _CH)])
        return carry
    lax.fori_loop(0, _ROWS_PER_TILE // _CH, zacc, 0)

    plsc.subcore_barrier()

    base_e = wid * _NCT * _CH

    def body(k, carry):
        e0 = pl.multiple_of(base_e + k * _CH, 8)
        pltpu.sync_copy(src_hbm.at[pl.ds(e0, _CH)], src2d)
        pltpu.sync_copy(dst_hbm.at[pl.ds(e0, _CH)], dst2d)
        pltpu.async_copy(support_hbm.at[src2d], rows_v, gsem).wait()
        pltpu.sync_copy(rows_v, acc_sh.at[dst2d], add=True)
        return carry
    lax.fori_loop(0, _NCT, body, 0)
    plsc.subcore_barrier()

    def ex(k, carry):
        r0 = base_r + k * _CH
        pltpu.sync_copy(acc_sh.at[pl.ds(r0, _CH)], rows_v)
        pltpu.sync_copy(rows_v, out_hbm.at[cid, pl.ds(r0, _CH)])
        return carry
    lax.fori_loop(0, _ROWS_PER_TILE // _CH, ex, 0)


@functools.cache
def _make_spmm():
    return pl.kernel(
        _spmm_body,
        out_type=jax.ShapeDtypeStruct((2, _NPAD, _NHID), jnp.float32),
        mesh=plsc.VectorSubcoreMesh(core_axis_name="c", subcore_axis_name="s"),
        scratch_types=[
            pltpu.VMEM((_CH,), jnp.int32),
            pltpu.VMEM((_CH,), jnp.int32),
            pltpu.VMEM((_CH, _NHID), jnp.float32),
            pltpu.VMEM_SHARED((_NPAD, _NHID), jnp.float32),
            pltpu.SemaphoreType.DMA,
        ],
    )


def _spmm(support, src, dst):
    npad_e = _EPAD - _E
    src_p = jnp.concatenate(
        [src, jnp.arange(npad_e, dtype=jnp.int32) % _N])
    dst_p = jnp.concatenate(
        [dst, _N + (jnp.arange(npad_e, dtype=jnp.int32) % (_NPAD - _N))])
    return _make_spmm()(support, src_p, dst_p)


def _dense_in_body(x_ref, w0_ref, wci_ref, bci_ref, sup_ref, ci_ref):
    x = x_ref[...]
    sup_ref[...] = jnp.dot(x, w0_ref[...], preferred_element_type=jnp.float32)
    ci_ref[...] = (jnp.dot(x, wci_ref[...], preferred_element_type=jnp.float32)
                   + bci_ref[...])


def _dense_in(x, W0, Wci, bci):
    full = pl.BlockSpec((_NHID, _NHID), lambda i: (0, 0))
    row = pl.BlockSpec((1, _NHID), lambda i: (0, 0))
    blk = pl.BlockSpec((_BLK, _NHID), lambda i: (i, 0))
    return pl.pallas_call(
        _dense_in_body,
        grid=(_N // _BLK,),
        in_specs=[blk, full, full, row],
        out_specs=[blk, blk],
        out_shape=[jax.ShapeDtypeStruct((_N, _NHID), jnp.float32),
                   jax.ShapeDtypeStruct((_N, _NHID), jnp.float32)],
    )(x, W0, Wci, bci)


def _gate(agg_a, agg_b, b, ci, x, wco, bco):
    out_x = agg_a[0] + agg_b[0] + b
    z = jax.nn.sigmoid(
        ci + jnp.dot(out_x, wco, preferred_element_type=jnp.float32) + bco)
    return z * out_x + (1.0 - z) * x


def _gate_next_body(agga_ref, aggb_ref, b_ref, ci_ref, x_ref, wco_ref,
                    bco_ref, wn_ref, out_ref):
    h = jax.nn.relu(_gate(agga_ref[...], aggb_ref[...], b_ref[...],
                          ci_ref[...], x_ref[...], wco_ref[...], bco_ref[...]))
    out_ref[...] = jnp.dot(h, wn_ref[...], preferred_element_type=jnp.float32)


def _gate_next(agg, b, ci, x, Wco, bco, Wnext):
    full = pl.BlockSpec((_NHID, _NHID), lambda i: (0, 0))
    row = pl.BlockSpec((1, _NHID), lambda i: (0, 0))
    blk = pl.BlockSpec((_BLK, _NHID), lambda i: (i, 0))
    agg0 = pl.BlockSpec((1, _BLK, _NHID), lambda i: (0, i, 0))
    agg1 = pl.BlockSpec((1, _BLK, _NHID), lambda i: (1, i, 0))
    return pl.pallas_call(
        _gate_next_body,
        grid=(_N // _BLK,),
        in_specs=[agg0, agg1, row, blk, blk, full, row, full],
        out_specs=blk,
        out_shape=jax.ShapeDtypeStruct((_N, _NHID), jnp.float32),
    )(agg, agg, b, ci, x, Wco, bco, Wnext)


def _gate_final_body(agga_ref, aggb_ref, b_ref, ci_ref, x_ref, wco_ref,
                     bco_ref, wf_ref, bf_ref, out_ref):
    h = _gate(agga_ref[...], aggb_ref[...], b_ref[...],
              ci_ref[...], x_ref[...], wco_ref[...], bco_ref[...])
    out_ref[...] = (jnp.dot(h, wf_ref[...], preferred_element_type=jnp.float32)
                    + bf_ref[...])


def _gate_final(agg, b, ci, x, Wco, bco, Wf, bf):
    full = pl.BlockSpec((_NHID, _NHID), lambda i: (0, 0))
    wf_spec = pl.BlockSpec((_NHID, _NCLASS), lambda i: (0, 0))
    row = pl.BlockSpec((1, _NHID), lambda i: (0, 0))
    rowf = pl.BlockSpec((1, _NCLASS), lambda i: (0, 0))
    blk = pl.BlockSpec((_BLK, _NHID), lambda i: (i, 0))
    blkf = pl.BlockSpec((_BLK, _NCLASS), lambda i: (i, 0))
    agg0 = pl.BlockSpec((1, _BLK, _NHID), lambda i: (0, i, 0))
    agg1 = pl.BlockSpec((1, _BLK, _NHID), lambda i: (1, i, 0))
    return pl.pallas_call(
        _gate_final_body,
        grid=(_N // _BLK,),
        in_specs=[agg0, agg1, row, blk, blk, full, row, wf_spec, rowf],
        out_specs=blkf,
        out_shape=jax.ShapeDtypeStruct((_N, _NCLASS), jnp.float32),
    )(agg, agg, b, ci, x, Wco, bco, Wf, bf)


def kernel(x, edge_index, W0, b0, W1, b1, W2, b2, Wci, bci, Wco, bco, Wf, bf):
    src = edge_index[0].astype(jnp.int32)
    dst = edge_index[1].astype(jnp.int32)
    bci2 = bci.reshape(1, _NHID)
    bco2 = bco.reshape(1, _NHID)
    bf2 = bf.reshape(1, _NCLASS)

    support0, ci = _dense_in(x, W0, Wci, bci2)
    agg0 = _spmm(support0, src, dst)
    support1 = _gate_next(agg0, b0, ci, x, Wco, bco2, W1)
    agg1 = _spmm(support1, src, dst)
    support2 = _gate_next(agg1, b1, ci, x, Wco, bco2, W2)
    agg2 = _spmm(support2, src, dst)
    return _gate_final(agg2, b2, ci, x, Wco, bco2, Wf, bf2)


# speedup vs baseline: 3.5343x; 1.3458x over previous
import functools

import jax
import jax.numpy as jnp
from jax import lax
from jax.experimental import pallas as pl
from jax.experimental.pallas import tpu as pltpu
from jax.experimental.pallas import tpu_sc as plsc

_N = 10000
_E = 320000
_NHID = 128
_NCLASS = 64

_CH = 128
_NCHUNK = _E // _CH
_NTILES = 32
_NPAD = 10240
_ROWS_PER_TILE = _NPAD // 16
_BLK = 1000


_NCT = 80
_EPAD = _NTILES * _NCT * _CH


def _spmm_body(support_hbm, src_hbm, dst_hbm, out_hbm,
               src2d, dst2d, rows_v, acc_sh, gsem):
    cid = lax.axis_index("c")
    sid = lax.axis_index("s")
    wid = sid * 2 + cid

    def zfill(i, carry):
        r = i // (_NHID // 16)
        c = (i % (_NHID // 16)) * 16
        rows_v[r, pl.ds(c, 16)] = jnp.zeros((16,), jnp.float32)
        return carry
    lax.fori_loop(0, _CH * (_NHID // 16), zfill, 0)

    base_r = sid * _ROWS_PER_TILE
    def zacc(k, carry):
        pltpu.sync_copy(rows_v, acc_sh.at[pl.ds(base_r + k * _CH, _CH)])
        return carry
    lax.fori_loop(0, _ROWS_PER_TILE // _CH, zacc, 0)

    pltpu.sync_copy(src_hbm.at[pl.ds(wid * _NCT, _NCT)], src2d)
    pltpu.sync_copy(dst_hbm.at[pl.ds(wid * _NCT, _NCT)], dst2d)
    plsc.subcore_barrier()

    def body(k, carry):
        pltpu.async_copy(support_hbm.at[src2d.at[k]], rows_v, gsem).wait()
        pltpu.sync_copy(rows_v, acc_sh.at[dst2d.at[k]], add=True)
        return carry
    lax.fori_loop(0, _NCT, body, 0)
    plsc.subcore_barrier()

    def ex(k, carry):
        r0 = base_r + k * _CH
        pltpu.sync_copy(acc_sh.at[pl.ds(r0, _CH)], rows_v)
        pltpu.sync_copy(rows_v, out_hbm.at[cid, pl.ds(r0, _CH)])
        return carry
    lax.fori_loop(0, _ROWS_PER_TILE // _CH, ex, 0)


@functools.cache
def _make_spmm():
    return pl.kernel(
        _spmm_body,
        out_type=jax.ShapeDtypeStruct((2, _NPAD, _NHID), jnp.float32),
        mesh=plsc.VectorSubcoreMesh(core_axis_name="c", subcore_axis_name="s"),
        scratch_types=[
            pltpu.VMEM((_NCT, _CH), jnp.int32),
            pltpu.VMEM((_NCT, _CH), jnp.int32),
            pltpu.VMEM((_CH, _NHID), jnp.float32),
            pltpu.VMEM_SHARED((_NPAD, _NHID), jnp.float32),
            pltpu.SemaphoreType.DMA,
        ],
    )


def _spmm(support, src, dst):
    npad_e = _EPAD - _E
    src_p = jnp.concatenate(
        [src, jnp.arange(npad_e, dtype=jnp.int32) % _N]).reshape(-1, _CH)
    dst_p = jnp.concatenate(
        [dst, _N + (jnp.arange(npad_e, dtype=jnp.int32) % (_NPAD - _N))]
    ).reshape(-1, _CH)
    return _make_spmm()(support, src_p, dst_p)


def _dense_in_body(x_ref, w0_ref, wci_ref, bci_ref, sup_ref, ci_ref):
    x = x_ref[...]
    sup_ref[...] = jnp.dot(x, w0_ref[...], preferred_element_type=jnp.float32)
    ci_ref[...] = (jnp.dot(x, wci_ref[...], preferred_element_type=jnp.float32)
                   + bci_ref[...])


def _dense_in(x, W0, Wci, bci):
    full = pl.BlockSpec((_NHID, _NHID), lambda i: (0, 0))
    row = pl.BlockSpec((1, _NHID), lambda i: (0, 0))
    blk = pl.BlockSpec((_BLK, _NHID), lambda i: (i, 0))
    return pl.pallas_call(
        _dense_in_body,
        grid=(_N // _BLK,),
        in_specs=[blk, full, full, row],
        out_specs=[blk, blk],
        out_shape=[jax.ShapeDtypeStruct((_N, _NHID), jnp.float32),
                   jax.ShapeDtypeStruct((_N, _NHID), jnp.float32)],
    )(x, W0, Wci, bci)


def _gate(agg_a, agg_b, b, ci, x, wco, bco):
    out_x = agg_a[0] + agg_b[0] + b
    z = jax.nn.sigmoid(
        ci + jnp.dot(out_x, wco, preferred_element_type=jnp.float32) + bco)
    return z * out_x + (1.0 - z) * x


def _gate_next_body(agga_ref, aggb_ref, b_ref, ci_ref, x_ref, wco_ref,
                    bco_ref, wn_ref, out_ref):
    h = jax.nn.relu(_gate(agga_ref[...], aggb_ref[...], b_ref[...],
                          ci_ref[...], x_ref[...], wco_ref[...], bco_ref[...]))
    out_ref[...] = jnp.dot(h, wn_ref[...], preferred_element_type=jnp.float32)


def _gate_next(agg, b, ci, x, Wco, bco, Wnext):
    full = pl.BlockSpec((_NHID, _NHID), lambda i: (0, 0))
    row = pl.BlockSpec((1, _NHID), lambda i: (0, 0))
    blk = pl.BlockSpec((_BLK, _NHID), lambda i: (i, 0))
    agg0 = pl.BlockSpec((1, _BLK, _NHID), lambda i: (0, i, 0))
    agg1 = pl.BlockSpec((1, _BLK, _NHID), lambda i: (1, i, 0))
    return pl.pallas_call(
        _gate_next_body,
        grid=(_N // _BLK,),
        in_specs=[agg0, agg1, row, blk, blk, full, row, full],
        out_specs=blk,
        out_shape=jax.ShapeDtypeStruct((_N, _NHID), jnp.float32),
    )(agg, agg, b, ci, x, Wco, bco, Wnext)


def _gate_final_body(agga_ref, aggb_ref, b_ref, ci_ref, x_ref, wco_ref,
                     bco_ref, wf_ref, bf_ref, out_ref):
    h = _gate(agga_ref[...], aggb_ref[...], b_ref[...],
              ci_ref[...], x_ref[...], wco_ref[...], bco_ref[...])
    out_ref[...] = (jnp.dot(h, wf_ref[...], preferred_element_type=jnp.float32)
                    + bf_ref[...])


def _gate_final(agg, b, ci, x, Wco, bco, Wf, bf):
    full = pl.BlockSpec((_NHID, _NHID), lambda i: (0, 0))
    wf_spec = pl.BlockSpec((_NHID, _NCLASS), lambda i: (0, 0))
    row = pl.BlockSpec((1, _NHID), lambda i: (0, 0))
    rowf = pl.BlockSpec((1, _NCLASS), lambda i: (0, 0))
    blk = pl.BlockSpec((_BLK, _NHID), lambda i: (i, 0))
    blkf = pl.BlockSpec((_BLK, _NCLASS), lambda i: (i, 0))
    agg0 = pl.BlockSpec((1, _BLK, _NHID), lambda i: (0, i, 0))
    agg1 = pl.BlockSpec((1, _BLK, _NHID), lambda i: (1, i, 0))
    return pl.pallas_call(
        _gate_final_body,
        grid=(_N // _BLK,),
        in_specs=[agg0, agg1, row, blk, blk, full, row, wf_spec, rowf],
        out_specs=blkf,
        out_shape=jax.ShapeDtypeStruct((_N, _NCLASS), jnp.float32),
    )(agg, agg, b, ci, x, Wco, bco, Wf, bf)


def kernel(x, edge_index, W0, b0, W1, b1, W2, b2, Wci, bci, Wco, bco, Wf, bf):
    src = edge_index[0].astype(jnp.int32)
    dst = edge_index[1].astype(jnp.int32)
    bci2 = bci.reshape(1, _NHID)
    bco2 = bco.reshape(1, _NHID)
    bf2 = bf.reshape(1, _NCLASS)

    support0, ci = _dense_in(x, W0, Wci, bci2)
    agg0 = _spmm(support0, src, dst)
    support1 = _gate_next(agg0, b0, ci, x, Wco, bco2, W1)
    agg1 = _spmm(support1, src, dst)
    support2 = _gate_next(agg1, b1, ci, x, Wco, bco2, W2)
    agg2 = _spmm(support2, src, dst)
    return _gate_final(agg2, b2, ci, x, Wco, bco2, Wf, bf2)


# speedup vs baseline: 3.9971x; 1.1309x over previous
import functools

import jax
import jax.numpy as jnp
from jax import lax
from jax.experimental import pallas as pl
from jax.experimental.pallas import tpu as pltpu
from jax.experimental.pallas import tpu_sc as plsc

_N = 10000
_E = 320000
_NHID = 128
_NCLASS = 64

_CH = 128
_NCHUNK = _E // _CH
_NTILES = 32
_NPAD = 10240
_ROWS_PER_TILE = _NPAD // 16
_BLK = 1000


_NCT = 80
_HALF = _NCT // 2
_EPAD = _NTILES * _NCT * _CH


def _spmm_body(support_hbm, src_hbm, dst_hbm, out_hbm,
               src2d, dst2d, rows0, rows1, acc_sh, gsem0, gsem1):
    rows_v = rows0
    cid = lax.axis_index("c")
    sid = lax.axis_index("s")
    wid = sid * 2 + cid

    def zfill(i, carry):
        r = i // (_NHID // 16)
        c = (i % (_NHID // 16)) * 16
        rows_v[r, pl.ds(c, 16)] = jnp.zeros((16,), jnp.float32)
        return carry
    lax.fori_loop(0, _CH * (_NHID // 16), zfill, 0)

    base_r = sid * _ROWS_PER_TILE
    def zacc(k, carry):
        pltpu.sync_copy(rows_v, acc_sh.at[pl.ds(base_r + k * _CH, _CH)])
        return carry
    lax.fori_loop(0, _ROWS_PER_TILE // _CH, zacc, 0)

    plsc.subcore_barrier()

    for h in range(_NCT // _HALF):
        pltpu.sync_copy(
            src_hbm.at[pl.ds(wid * _NCT + h * _HALF, _HALF)], src2d)
        pltpu.sync_copy(
            dst_hbm.at[pl.ds(wid * _NCT + h * _HALF, _HALF)], dst2d)

        def body2(g, carry):
            k0 = 2 * g
            k1 = 2 * g + 1
            d0 = pltpu.async_copy(support_hbm.at[src2d.at[k0]], rows0, gsem0)
            d1 = pltpu.async_copy(support_hbm.at[src2d.at[k1]], rows1, gsem1)
            d0.wait()
            pltpu.sync_copy(rows0, acc_sh.at[dst2d.at[k0]], add=True)
            d1.wait()
            pltpu.sync_copy(rows1, acc_sh.at[dst2d.at[k1]], add=True)
            return carry
        lax.fori_loop(0, _HALF // 2, body2, 0)
    plsc.subcore_barrier()

    def ex(k, carry):
        r0 = base_r + k * _CH
        pltpu.sync_copy(acc_sh.at[pl.ds(r0, _CH)], rows_v)
        pltpu.sync_copy(rows_v, out_hbm.at[cid, pl.ds(r0, _CH)])
        return carry
    lax.fori_loop(0, _ROWS_PER_TILE // _CH, ex, 0)


@functools.cache
def _make_spmm():
    return pl.kernel(
        _spmm_body,
        out_type=jax.ShapeDtypeStruct((2, _NPAD, _NHID), jnp.float32),
        mesh=plsc.VectorSubcoreMesh(core_axis_name="c", subcore_axis_name="s"),
        scratch_types=[
            pltpu.VMEM((_HALF, _CH), jnp.int32),
            pltpu.VMEM((_HALF, _CH), jnp.int32),
            pltpu.VMEM((_CH, _NHID), jnp.float32),
            pltpu.VMEM((_CH, _NHID), jnp.float32),
            pltpu.VMEM_SHARED((_NPAD, _NHID), jnp.float32),
            pltpu.SemaphoreType.DMA,
            pltpu.SemaphoreType.DMA,
        ],
    )


def _spmm(support, src, dst):
    npad_e = _EPAD - _E
    src_p = jnp.concatenate(
        [src, jnp.arange(npad_e, dtype=jnp.int32) % _N]).reshape(-1, _CH)
    dst_p = jnp.concatenate(
        [dst, _N + (jnp.arange(npad_e, dtype=jnp.int32) % (_NPAD - _N))]
    ).reshape(-1, _CH)
    return _make_spmm()(support, src_p, dst_p)


def _dense_in_body(x_ref, w0_ref, wci_ref, bci_ref, sup_ref, ci_ref):
    x = x_ref[...]
    sup_ref[...] = jnp.dot(x, w0_ref[...], preferred_element_type=jnp.float32)
    ci_ref[...] = (jnp.dot(x, wci_ref[...], preferred_element_type=jnp.float32)
                   + bci_ref[...])


def _dense_in(x, W0, Wci, bci):
    full = pl.BlockSpec((_NHID, _NHID), lambda i: (0, 0))
    row = pl.BlockSpec((1, _NHID), lambda i: (0, 0))
    blk = pl.BlockSpec((_BLK, _NHID), lambda i: (i, 0))
    return pl.pallas_call(
        _dense_in_body,
        grid=(_N // _BLK,),
        in_specs=[blk, full, full, row],
        out_specs=[blk, blk],
        out_shape=[jax.ShapeDtypeStruct((_N, _NHID), jnp.float32),
                   jax.ShapeDtypeStruct((_N, _NHID), jnp.float32)],
    )(x, W0, Wci, bci)


def _gate(agg_a, agg_b, b, ci, x, wco, bco):
    out_x = agg_a[0] + agg_b[0] + b
    z = jax.nn.sigmoid(
        ci + jnp.dot(out_x, wco, preferred_element_type=jnp.float32) + bco)
    return z * out_x + (1.0 - z) * x


def _gate_next_body(agga_ref, aggb_ref, b_ref, ci_ref, x_ref, wco_ref,
                    bco_ref, wn_ref, out_ref):
    h = jax.nn.relu(_gate(agga_ref[...], aggb_ref[...], b_ref[...],
                          ci_ref[...], x_ref[...], wco_ref[...], bco_ref[...]))
    out_ref[...] = jnp.dot(h, wn_ref[...], preferred_element_type=jnp.float32)


def _gate_next(agg, b, ci, x, Wco, bco, Wnext):
    full = pl.BlockSpec((_NHID, _NHID), lambda i: (0, 0))
    row = pl.BlockSpec((1, _NHID), lambda i: (0, 0))
    blk = pl.BlockSpec((_BLK, _NHID), lambda i: (i, 0))
    agg0 = pl.BlockSpec((1, _BLK, _NHID), lambda i: (0, i, 0))
    agg1 = pl.BlockSpec((1, _BLK, _NHID), lambda i: (1, i, 0))
    return pl.pallas_call(
        _gate_next_body,
        grid=(_N // _BLK,),
        in_specs=[agg0, agg1, row, blk, blk, full, row, full],
        out_specs=blk,
        out_shape=jax.ShapeDtypeStruct((_N, _NHID), jnp.float32),
    )(agg, agg, b, ci, x, Wco, bco, Wnext)


def _gate_final_body(agga_ref, aggb_ref, b_ref, ci_ref, x_ref, wco_ref,
                     bco_ref, wf_ref, bf_ref, out_ref):
    h = _gate(agga_ref[...], aggb_ref[...], b_ref[...],
              ci_ref[...], x_ref[...], wco_ref[...], bco_ref[...])
    out_ref[...] = (jnp.dot(h, wf_ref[...], preferred_element_type=jnp.float32)
                    + bf_ref[...])


def _gate_final(agg, b, ci, x, Wco, bco, Wf, bf):
    full = pl.BlockSpec((_NHID, _NHID), lambda i: (0, 0))
    wf_spec = pl.BlockSpec((_NHID, _NCLASS), lambda i: (0, 0))
    row = pl.BlockSpec((1, _NHID), lambda i: (0, 0))
    rowf = pl.BlockSpec((1, _NCLASS), lambda i: (0, 0))
    blk = pl.BlockSpec((_BLK, _NHID), lambda i: (i, 0))
    blkf = pl.BlockSpec((_BLK, _NCLASS), lambda i: (i, 0))
    agg0 = pl.BlockSpec((1, _BLK, _NHID), lambda i: (0, i, 0))
    agg1 = pl.BlockSpec((1, _BLK, _NHID), lambda i: (1, i, 0))
    return pl.pallas_call(
        _gate_final_body,
        grid=(_N // _BLK,),
        in_specs=[agg0, agg1, row, blk, blk, full, row, wf_spec, rowf],
        out_specs=blkf,
        out_shape=jax.ShapeDtypeStruct((_N, _NCLASS), jnp.float32),
    )(agg, agg, b, ci, x, Wco, bco, Wf, bf)


def kernel(x, edge_index, W0, b0, W1, b1, W2, b2, Wci, bci, Wco, bco, Wf, bf):
    src = edge_index[0].astype(jnp.int32)
    dst = edge_index[1].astype(jnp.int32)
    bci2 = bci.reshape(1, _NHID)
    bco2 = bco.reshape(1, _NHID)
    bf2 = bf.reshape(1, _NCLASS)

    support0, ci = _dense_in(x, W0, Wci, bci2)
    agg0 = _spmm(support0, src, dst)
    support1 = _gate_next(agg0, b0, ci, x, Wco, bco2, W1)
    agg1 = _spmm(support1, src, dst)
    support2 = _gate_next(agg1, b1, ci, x, Wco, bco2, W2)
    agg2 = _spmm(support2, src, dst)
    return _gate_final(agg2, b2, ci, x, Wco, bco2, Wf, bf2)


# speedup vs baseline: 5.1957x; 1.2998x over previous
import functools

import jax
import jax.numpy as jnp
from jax import lax
from jax.experimental import pallas as pl
from jax.experimental.pallas import tpu as pltpu
from jax.experimental.pallas import tpu_sc as plsc

_N = 10000
_E = 320000
_NHID = 128
_NCLASS = 64

_CH = 128
_NCHUNK = _E // _CH
_NTILES = 32
_NPAD = 10240
_ROWS_PER_TILE = _NPAD // 16
_BLK = 1000


_NCT = 80
_HALF = _NCT // 2
_EPAD = _NTILES * _NCT * _CH


def _spmm_body(support_hbm, src_hbm, dst_hbm, out_hbm,
               src2d, dst2d, rows0, rows1, acc_sh, gsem0, gsem1):
    rows_v = rows0
    cid = lax.axis_index("c")
    sid = lax.axis_index("s")
    wid = sid * 2 + cid

    def zfill(i, carry):
        r = i // (_NHID // 16)
        c = (i % (_NHID // 16)) * 16
        rows_v[r, pl.ds(c, 16)] = jnp.zeros((16,), jnp.float32)
        return carry
    lax.fori_loop(0, _CH * (_NHID // 16), zfill, 0)

    base_r = sid * _ROWS_PER_TILE
    def zacc(k, carry):
        pltpu.sync_copy(rows_v, acc_sh.at[pl.ds(base_r + k * _CH, _CH)])
        return carry
    lax.fori_loop(0, _ROWS_PER_TILE // _CH, zacc, 0)

    plsc.subcore_barrier()

    def wait_gather(b_rows, b_sem):
        pltpu.make_async_copy(support_hbm.at[src2d.at[0]], b_rows,
                              b_sem).wait()

    for h in range(_NCT // _HALF):
        pltpu.sync_copy(
            src_hbm.at[pl.ds(wid * _NCT + h * _HALF, _HALF)], src2d)
        pltpu.sync_copy(
            dst_hbm.at[pl.ds(wid * _NCT + h * _HALF, _HALF)], dst2d)

        pltpu.async_copy(support_hbm.at[src2d.at[0]], rows0, gsem0)
        pltpu.async_copy(support_hbm.at[src2d.at[1]], rows1, gsem1)

        def body2(g, carry):
            k0 = 2 * g
            wait_gather(rows0, gsem0)
            pltpu.sync_copy(rows0, acc_sh.at[dst2d.at[k0]], add=True)
            pltpu.async_copy(support_hbm.at[src2d.at[k0 + 2]], rows0, gsem0)
            wait_gather(rows1, gsem1)
            pltpu.sync_copy(rows1, acc_sh.at[dst2d.at[k0 + 1]], add=True)
            pltpu.async_copy(support_hbm.at[src2d.at[k0 + 3]], rows1, gsem1)
            return carry
        lax.fori_loop(0, _HALF // 2 - 1, body2, 0)

        wait_gather(rows0, gsem0)
        pltpu.sync_copy(rows0, acc_sh.at[dst2d.at[_HALF - 2]], add=True)
        wait_gather(rows1, gsem1)
        pltpu.sync_copy(rows1, acc_sh.at[dst2d.at[_HALF - 1]], add=True)
    plsc.subcore_barrier()

    def ex(k, carry):
        r0 = base_r + k * _CH
        pltpu.sync_copy(acc_sh.at[pl.ds(r0, _CH)], rows_v)
        pltpu.sync_copy(rows_v, out_hbm.at[cid, pl.ds(r0, _CH)])
        return carry
    lax.fori_loop(0, _ROWS_PER_TILE // _CH, ex, 0)


@functools.cache
def _make_spmm():
    return pl.kernel(
        _spmm_body,
        out_type=jax.ShapeDtypeStruct((2, _NPAD, _NHID), jnp.float32),
        mesh=plsc.VectorSubcoreMesh(core_axis_name="c", subcore_axis_name="s"),
        scratch_types=[
            pltpu.VMEM((_HALF, _CH), jnp.int32),
            pltpu.VMEM((_HALF, _CH), jnp.int32),
            pltpu.VMEM((_CH, _NHID), jnp.float32),
            pltpu.VMEM((_CH, _NHID), jnp.float32),
            pltpu.VMEM_SHARED((_NPAD, _NHID), jnp.float32),
            pltpu.SemaphoreType.DMA,
            pltpu.SemaphoreType.DMA,
        ],
    )


def _spmm(support, src, dst):
    npad_e = _EPAD - _E
    src_p = jnp.concatenate(
        [src, jnp.arange(npad_e, dtype=jnp.int32) % _N]).reshape(-1, _CH)
    dst_p = jnp.concatenate(
        [dst, _N + (jnp.arange(npad_e, dtype=jnp.int32) % (_NPAD - _N))]
    ).reshape(-1, _CH)
    return _make_spmm()(support, src_p, dst_p)


def _dense_in_body(x_ref, w0_ref, wci_ref, bci_ref, sup_ref, ci_ref):
    x = x_ref[...]
    sup_ref[...] = jnp.dot(x, w0_ref[...], preferred_element_type=jnp.float32)
    ci_ref[...] = (jnp.dot(x, wci_ref[...], preferred_element_type=jnp.float32)
                   + bci_ref[...])


def _dense_in(x, W0, Wci, bci):
    full = pl.BlockSpec((_NHID, _NHID), lambda i: (0, 0))
    row = pl.BlockSpec((1, _NHID), lambda i: (0, 0))
    blk = pl.BlockSpec((_BLK, _NHID), lambda i: (i, 0))
    return pl.pallas_call(
        _dense_in_body,
        grid=(_N // _BLK,),
        in_specs=[blk, full, full, row],
        out_specs=[blk, blk],
        out_shape=[jax.ShapeDtypeStruct((_N, _NHID), jnp.float32),
                   jax.ShapeDtypeStruct((_N, _NHID), jnp.float32)],
    )(x, W0, Wci, bci)


def _gate(agg_a, agg_b, b, ci, x, wco, bco):
    out_x = agg_a[0] + agg_b[0] + b
    z = jax.nn.sigmoid(
        ci + jnp.dot(out_x, wco, preferred_element_type=jnp.float32) + bco)
    return z * out_x + (1.0 - z) * x


def _gate_next_body(agga_ref, aggb_ref, b_ref, ci_ref, x_ref, wco_ref,
                    bco_ref, wn_ref, out_ref):
    h = jax.nn.relu(_gate(agga_ref[...], aggb_ref[...], b_ref[...],
                          ci_ref[...], x_ref[...], wco_ref[...], bco_ref[...]))
    out_ref[...] = jnp.dot(h, wn_ref[...], preferred_element_type=jnp.float32)


def _gate_next(agg, b, ci, x, Wco, bco, Wnext):
    full = pl.BlockSpec((_NHID, _NHID), lambda i: (0, 0))
    row = pl.BlockSpec((1, _NHID), lambda i: (0, 0))
    blk = pl.BlockSpec((_BLK, _NHID), lambda i: (i, 0))
    agg0 = pl.BlockSpec((1, _BLK, _NHID), lambda i: (0, i, 0))
    agg1 = pl.BlockSpec((1, _BLK, _NHID), lambda i: (1, i, 0))
    return pl.pallas_call(
        _gate_next_body,
        grid=(_N // _BLK,),
        in_specs=[agg0, agg1, row, blk, blk, full, row, full],
        out_specs=blk,
        out_shape=jax.ShapeDtypeStruct((_N, _NHID), jnp.float32),
    )(agg, agg, b, ci, x, Wco, bco, Wnext)


def _gate_final_body(agga_ref, aggb_ref, b_ref, ci_ref, x_ref, wco_ref,
                     bco_ref, wf_ref, bf_ref, out_ref):
    h = _gate(agga_ref[...], aggb_ref[...], b_ref[...],
              ci_ref[...], x_ref[...], wco_ref[...], bco_ref[...])
    out_ref[...] = (jnp.dot(h, wf_ref[...], preferred_element_type=jnp.float32)
                    + bf_ref[...])


def _gate_final(agg, b, ci, x, Wco, bco, Wf, bf):
    full = pl.BlockSpec((_NHID, _NHID), lambda i: (0, 0))
    wf_spec = pl.BlockSpec((_NHID, _NCLASS), lambda i: (0, 0))
    row = pl.BlockSpec((1, _NHID), lambda i: (0, 0))
    rowf = pl.BlockSpec((1, _NCLASS), lambda i: (0, 0))
    blk = pl.BlockSpec((_BLK, _NHID), lambda i: (i, 0))
    blkf = pl.BlockSpec((_BLK, _NCLASS), lambda i: (i, 0))
    agg0 = pl.BlockSpec((1, _BLK, _NHID), lambda i: (0, i, 0))
    agg1 = pl.BlockSpec((1, _BLK, _NHID), lambda i: (1, i, 0))
    return pl.pallas_call(
        _gate_final_body,
        grid=(_N // _BLK,),
        in_specs=[agg0, agg1, row, blk, blk, full, row, wf_spec, rowf],
        out_specs=blkf,
        out_shape=jax.ShapeDtypeStruct((_N, _NCLASS), jnp.float32),
    )(agg, agg, b, ci, x, Wco, bco, Wf, bf)


def kernel(x, edge_index, W0, b0, W1, b1, W2, b2, Wci, bci, Wco, bco, Wf, bf):
    src = edge_index[0].astype(jnp.int32)
    dst = edge_index[1].astype(jnp.int32)
    bci2 = bci.reshape(1, _NHID)
    bco2 = bco.reshape(1, _NHID)
    bf2 = bf.reshape(1, _NCLASS)

    support0, ci = _dense_in(x, W0, Wci, bci2)
    agg0 = _spmm(support0, src, dst)
    support1 = _gate_next(agg0, b0, ci, x, Wco, bco2, W1)
    agg1 = _spmm(support1, src, dst)
    support2 = _gate_next(agg1, b1, ci, x, Wco, bco2, W2)
    agg2 = _spmm(support2, src, dst)
    return _gate_final(agg2, b2, ci, x, Wco, bco2, Wf, bf2)


# speedup vs baseline: 5.3451x; 1.0288x over previous
import functools

import jax
import jax.numpy as jnp
from jax import lax
from jax.experimental import pallas as pl
from jax.experimental.pallas import tpu as pltpu
from jax.experimental.pallas import tpu_sc as plsc

_N = 10000
_E = 320000
_NHID = 128
_NCLASS = 64

_CH = 128
_NCHUNK = _E // _CH
_NTILES = 32
_NPAD = 10240
_ROWS_PER_TILE = _NPAD // 16
_BLK = 1000


_NCT = 80
_HALF = _NCT // 2
_EPAD = _NTILES * _NCT * _CH


def _spmm_body(support_hbm, src_hbm, dst_hbm, out_hbm,
               src2d, dst2d, rows0, rows1, acc_sh, gsem0, gsem1):
    rows_v = rows0
    cid = lax.axis_index("c")
    sid = lax.axis_index("s")
    wid = sid * 2 + cid

    def zfill(i, carry):
        r = i // (_NHID // 16)
        c = (i % (_NHID // 16)) * 16
        rows_v[r, pl.ds(c, 16)] = jnp.zeros((16,), jnp.float32)
        return carry
    lax.fori_loop(0, _CH * (_NHID // 16), zfill, 0)

    base_r = sid * _ROWS_PER_TILE
    def zacc(k, carry):
        pltpu.sync_copy(rows_v, acc_sh.at[pl.ds(base_r + k * _CH, _CH)])
        return carry
    lax.fori_loop(0, _ROWS_PER_TILE // _CH, zacc, 0)

    plsc.subcore_barrier()

    def wait_gather(b_rows, b_sem):
        pltpu.make_async_copy(support_hbm.at[src2d.at[0]], b_rows,
                              b_sem).wait()

    for h in range(_NCT // _HALF):
        pltpu.sync_copy(
            src_hbm.at[pl.ds(wid * _NCT + h * _HALF, _HALF)], src2d)
        pltpu.sync_copy(
            dst_hbm.at[pl.ds(wid * _NCT + h * _HALF, _HALF)], dst2d)

        pltpu.async_copy(support_hbm.at[src2d.at[0]], rows0, gsem0)
        pltpu.async_copy(support_hbm.at[src2d.at[1]], rows1, gsem1)

        def body2(g, carry):
            k0 = 2 * g
            wait_gather(rows0, gsem0)
            pltpu.sync_copy(rows0, acc_sh.at[dst2d.at[k0]], add=True)
            pltpu.async_copy(support_hbm.at[src2d.at[k0 + 2]], rows0, gsem0)
            wait_gather(rows1, gsem1)
            pltpu.sync_copy(rows1, acc_sh.at[dst2d.at[k0 + 1]], add=True)
            pltpu.async_copy(support_hbm.at[src2d.at[k0 + 3]], rows1, gsem1)
            return carry
        lax.fori_loop(0, _HALF // 2 - 1, body2, 0)

        wait_gather(rows0, gsem0)
        pltpu.sync_copy(rows0, acc_sh.at[dst2d.at[_HALF - 2]], add=True)
        wait_gather(rows1, gsem1)
        pltpu.sync_copy(rows1, acc_sh.at[dst2d.at[_HALF - 1]], add=True)
    plsc.subcore_barrier()

    pltpu.sync_copy(acc_sh.at[pl.ds(base_r, _ROWS_PER_TILE)],
                    out_hbm.at[cid, pl.ds(base_r, _ROWS_PER_TILE)])


@functools.cache
def _make_spmm():
    return pl.kernel(
        _spmm_body,
        out_type=jax.ShapeDtypeStruct((2, _NPAD, _NHID), jnp.float32),
        mesh=plsc.VectorSubcoreMesh(core_axis_name="c", subcore_axis_name="s"),
        scratch_types=[
            pltpu.VMEM((_HALF, _CH), jnp.int32),
            pltpu.VMEM((_HALF, _CH), jnp.int32),
            pltpu.VMEM((_CH, _NHID), jnp.float32),
            pltpu.VMEM((_CH, _NHID), jnp.float32),
            pltpu.VMEM_SHARED((_NPAD, _NHID), jnp.float32),
            pltpu.SemaphoreType.DMA,
            pltpu.SemaphoreType.DMA,
        ],
    )


def _spmm(support, src, dst):
    npad_e = _EPAD - _E
    src_p = jnp.concatenate(
        [src, jnp.arange(npad_e, dtype=jnp.int32) % _N]).reshape(-1, _CH)
    dst_p = jnp.concatenate(
        [dst, _N + (jnp.arange(npad_e, dtype=jnp.int32) % (_NPAD - _N))]
    ).reshape(-1, _CH)
    return _make_spmm()(support, src_p, dst_p)


def _gate(agg_a, agg_b, w, b, x, wci, bci, wco, bco):
    out_x = jnp.dot(agg_a[0] + agg_b[0], w,
                    preferred_element_type=jnp.float32) + b
    ci = jnp.dot(x, wci, preferred_element_type=jnp.float32) + bci
    z = jax.nn.sigmoid(
        ci + jnp.dot(out_x, wco, preferred_element_type=jnp.float32) + bco)
    return z * out_x + (1.0 - z) * x


def _gate_mm_body(agga_ref, aggb_ref, w_ref, b_ref, x_ref, wci_ref, bci_ref,
                  wco_ref, bco_ref, out_ref):
    out_ref[...] = jax.nn.relu(
        _gate(agga_ref[...], aggb_ref[...], w_ref[...], b_ref[...],
              x_ref[...], wci_ref[...], bci_ref[...], wco_ref[...],
              bco_ref[...]))


def _gate_mm(agg, W, b, x, Wci, bci, Wco, bco):
    full = pl.BlockSpec((_NHID, _NHID), lambda i: (0, 0))
    row = pl.BlockSpec((1, _NHID), lambda i: (0, 0))
    blk = pl.BlockSpec((_BLK, _NHID), lambda i: (i, 0))
    agg0 = pl.BlockSpec((1, _BLK, _NHID), lambda i: (0, i, 0))
    agg1 = pl.BlockSpec((1, _BLK, _NHID), lambda i: (1, i, 0))
    return pl.pallas_call(
        _gate_mm_body,
        grid=(_N // _BLK,),
        in_specs=[agg0, agg1, full, row, blk, full, row, full, row],
        out_specs=blk,
        out_shape=jax.ShapeDtypeStruct((_N, _NHID), jnp.float32),
    )(agg, agg, W, b, x, Wci, bci, Wco, bco)


def _gate_final_body(agga_ref, aggb_ref, w_ref, b_ref, x_ref, wci_ref,
                     bci_ref, wco_ref, bco_ref, wf_ref, bf_ref, out_ref):
    h = _gate(agga_ref[...], aggb_ref[...], w_ref[...], b_ref[...],
              x_ref[...], wci_ref[...], bci_ref[...], wco_ref[...],
              bco_ref[...])
    out_ref[...] = (jnp.dot(h, wf_ref[...], preferred_element_type=jnp.float32)
                    + bf_ref[...])


def _gate_final(agg, W, b, x, Wci, bci, Wco, bco, Wf, bf):
    full = pl.BlockSpec((_NHID, _NHID), lambda i: (0, 0))
    wf_spec = pl.BlockSpec((_NHID, _NCLASS), lambda i: (0, 0))
    row = pl.BlockSpec((1, _NHID), lambda i: (0, 0))
    rowf = pl.BlockSpec((1, _NCLASS), lambda i: (0, 0))
    blk = pl.BlockSpec((_BLK, _NHID), lambda i: (i, 0))
    blkf = pl.BlockSpec((_BLK, _NCLASS), lambda i: (i, 0))
    agg0 = pl.BlockSpec((1, _BLK, _NHID), lambda i: (0, i, 0))
    agg1 = pl.BlockSpec((1, _BLK, _NHID), lambda i: (1, i, 0))
    return pl.pallas_call(
        _gate_final_body,
        grid=(_N // _BLK,),
        in_specs=[agg0, agg1, full, row, blk, full, row, full, row,
                  wf_spec, rowf],
        out_specs=blkf,
        out_shape=jax.ShapeDtypeStruct((_N, _NCLASS), jnp.float32),
    )(agg, agg, W, b, x, Wci, bci, Wco, bco, Wf, bf)


def kernel(x, edge_index, W0, b0, W1, b1, W2, b2, Wci, bci, Wco, bco, Wf, bf):
    src = edge_index[0].astype(jnp.int32)
    dst = edge_index[1].astype(jnp.int32)
    bci2 = bci.reshape(1, _NHID)
    bco2 = bco.reshape(1, _NHID)
    bf2 = bf.reshape(1, _NCLASS)

    agg0 = _spmm(x, src, dst)
    h1 = _gate_mm(agg0, W0, b0, x, Wci, bci2, Wco, bco2)
    agg1 = _spmm(h1, src, dst)
    h2 = _gate_mm(agg1, W1, b1, x, Wci, bci2, Wco, bco2)
    agg2 = _spmm(h2, src, dst)
    return _gate_final(agg2, W2, b2, x, Wci, bci2, Wco, bco2, Wf, bf2)
